# baseline (speedup 1.0000x reference)
import functools

import jax
import jax.numpy as jnp
from jax import lax
from jax.experimental import pallas as pl
from jax.experimental.pallas import tpu as pltpu
from jax.experimental.pallas import tpu_sc as plsc

B = 16384
FACTORS = 64
NW = 32
BPW = B // NW

_mesh = plsc.VectorSubcoreMesh(core_axis_name="c", subcore_axis_name="s")


@functools.partial(
    pl.kernel,
    out_type=(
        jax.ShapeDtypeStruct((B,), jnp.float32),
        jax.ShapeDtypeStruct((B, FACTORS), jnp.float32),
        jax.ShapeDtypeStruct((B, FACTORS), jnp.float32),
    ),
    mesh=_mesh,
    compiler_params=pltpu.CompilerParams(use_tc_tiling_on_sc=False),
    scratch_types=[
        pltpu.VMEM((BPW,), jnp.int32),
        pltpu.VMEM((BPW,), jnp.int32),
        pltpu.VMEM((BPW, FACTORS), jnp.float32),
        pltpu.VMEM((BPW, FACTORS), jnp.float32),
        pltpu.VMEM((BPW,), jnp.float32),
        pltpu.SemaphoreType.DMA,
        pltpu.SemaphoreType.DMA,
    ],
)
def _gather_sc(user_hbm, item_hbm, bi_hbm, gu_hbm, gi_hbm,
               beta_hbm, gu_out_hbm, gi_out_hbm,
               uidx_v, iidx_v, gu_v, gi_v, bi_v, sem_in, sem_out):
    wid = lax.axis_index("s") * 2 + lax.axis_index("c")
    base = wid * BPW

    pltpu.sync_copy(user_hbm.at[pl.ds(base, BPW)], uidx_v)
    pltpu.sync_copy(item_hbm.at[pl.ds(base, BPW)], iidx_v)

    cp_gu = pltpu.async_copy(gu_hbm.at[uidx_v], gu_v, sem_in)
    cp_gi = pltpu.async_copy(gi_hbm.at[iidx_v], gi_v, sem_in)
    cp_bi = pltpu.async_copy(bi_hbm.at[iidx_v], bi_v, sem_in)
    cp_gu.wait()
    out_gu = pltpu.async_copy(gu_v, gu_out_hbm.at[pl.ds(base, BPW)], sem_out)
    cp_gi.wait()
    out_gi = pltpu.async_copy(gi_v, gi_out_hbm.at[pl.ds(base, BPW)], sem_out)
    cp_bi.wait()
    out_bi = pltpu.async_copy(bi_v, beta_hbm.at[pl.ds(base, BPW)], sem_out)
    out_gu.wait()
    out_gi.wait()
    out_bi.wait()


def _dot_tc_body(beta_ref, gu_ref, gi_ref, xui_ref):
    xui_ref[...] = beta_ref[...] + jnp.sum(gu_ref[...] * gi_ref[...], axis=1)


_dot_tc = pl.pallas_call(
    _dot_tc_body,
    out_shape=jax.ShapeDtypeStruct((B,), jnp.float32),
)


def kernel(user, item, Bi, Gu, Gi):
    user = user.astype(jnp.int32)
    item = item.astype(jnp.int32)
    beta_i, gamma_u, gamma_i = _gather_sc(user, item, Bi, Gu, Gi)
    xui = _dot_tc(beta_i, gamma_u, gamma_i)
    return (xui, beta_i, gamma_u, gamma_i)


# speedup vs baseline: 1.5242x; 1.5242x over previous
import functools

import jax
import jax.numpy as jnp
from jax import lax
from jax.experimental import pallas as pl
from jax.experimental.pallas import tpu as pltpu
from jax.experimental.pallas import tpu_sc as plsc

B = 16384
FACTORS = 64
NW = 32
BPW = B // NW
CH = 128
NCH = BPW // CH

_mesh = plsc.VectorSubcoreMesh(core_axis_name="c", subcore_axis_name="s")


@functools.partial(
    pl.kernel,
    out_type=(
        jax.ShapeDtypeStruct((B,), jnp.float32),
        jax.ShapeDtypeStruct((B, FACTORS), jnp.float32),
        jax.ShapeDtypeStruct((B, FACTORS), jnp.float32),
    ),
    mesh=_mesh,
    scratch_types=[
        pltpu.VMEM((BPW,), jnp.int32),
        pltpu.VMEM((BPW,), jnp.int32),
        pltpu.VMEM((2, CH, FACTORS), jnp.float32),
        pltpu.VMEM((2, CH, FACTORS), jnp.float32),
        pltpu.VMEM((BPW,), jnp.float32),
        pltpu.SemaphoreType.DMA,
        pltpu.SemaphoreType.DMA,
    ],
)
def _gather_sc(user_hbm, item_hbm, bi_hbm, gu_hbm, gi_hbm,
               beta_hbm, gu_out_hbm, gi_out_hbm,
               uidx_v, iidx_v, gu_v, gi_v, bi_v, sem_in, sem_out):
    wid = lax.axis_index("s") * 2 + lax.axis_index("c")
    base = wid * BPW

    pltpu.sync_copy(user_hbm.at[pl.ds(base, BPW)], uidx_v)
    pltpu.sync_copy(item_hbm.at[pl.ds(base, BPW)], iidx_v)

    cp_bi = pltpu.async_copy(bi_hbm.at[iidx_v], bi_v, sem_in)

    def make_group_body(c, buf):
        def group_body(g, _):
            j0 = g * 16
            uvec = uidx_v[pl.ds(c * CH + j0, 16)]
            ivec = iidx_v[pl.ds(c * CH + j0, 16)]
            cps = []
            for jj in range(16):
                cps.append(pltpu.async_copy(
                    gu_hbm.at[uvec[jj]], gu_v.at[buf, j0 + jj], sem_in))
                cps.append(pltpu.async_copy(
                    gi_hbm.at[ivec[jj]], gi_v.at[buf, j0 + jj], sem_in))
            for cp in cps:
                cp.wait()
            return 0
        return group_body

    out_cps = []
    for c in range(NCH):
        buf = c % 2
        if c >= 2:
            out_cps[2 * (c - 2)].wait()
            out_cps[2 * (c - 2) + 1].wait()
        lax.fori_loop(0, CH // 16, make_group_body(c, buf), 0)
        out_cps.append(pltpu.async_copy(
            gu_v.at[buf], gu_out_hbm.at[pl.ds(base + c * CH, CH)], sem_out))
        out_cps.append(pltpu.async_copy(
            gi_v.at[buf], gi_out_hbm.at[pl.ds(base + c * CH, CH)], sem_out))

    cp_bi.wait()
    out_bi = pltpu.async_copy(bi_v, beta_hbm.at[pl.ds(base, BPW)], sem_out)
    for cp in out_cps[2 * (NCH - 2):]:
        cp.wait()
    out_bi.wait()


def _dot_tc_body(beta_ref, gu_ref, gi_ref, xui_ref):
    xui_ref[...] = beta_ref[...] + jnp.sum(gu_ref[...] * gi_ref[...], axis=1)


_dot_tc = pl.pallas_call(
    _dot_tc_body,
    out_shape=jax.ShapeDtypeStruct((B,), jnp.float32),
)


def kernel(user, item, Bi, Gu, Gi):
    user = user.astype(jnp.int32)
    item = item.astype(jnp.int32)
    beta_i, gamma_u, gamma_i = _gather_sc(user, item, Bi, Gu, Gi)
    xui = _dot_tc(beta_i, gamma_u, gamma_i)
    return (xui, beta_i, gamma_u, gamma_i)
